# baseline (speedup 1.0000x reference)
import functools
import numpy as np
import jax
import jax.numpy as jnp
from jax import lax
from jax.experimental import pallas as pl
from jax.experimental.pallas import tpu as pltpu
from jax.experimental.pallas import tpu_sc as plsc

_D = 64
_MAX_LEN = 512
_NUM_CORES = 2
_NUM_SUBCORES = 16
_NW = _NUM_CORES * _NUM_SUBCORES
_L16 = 16


def _make_pe2(max_len, d_model):
    position = np.arange(max_len, dtype=np.float32)[:, None]
    div_term = np.exp(
        np.arange(0, d_model, 2, dtype=np.float32) * -(np.log(10000.0) / d_model)
    )
    pe = np.zeros((max_len, d_model), dtype=np.float32)
    pe[:, 0::2] = np.sin(position * div_term)
    pe[:, 1::2] = np.cos(position * div_term)
    return pe * 2.0


_MESH = dict(
    core_axis_name="c", subcore_axis_name="s",
    num_cores=_NUM_CORES, num_subcores=_NUM_SUBCORES,
)
_TC_TILED = pltpu.CompilerParams(use_tc_tiling_on_sc=True,
                                 needs_layout_passes=False,
                                 disable_bounds_checks=True)


def _wid():
    return lax.axis_index("s") * _NUM_CORES + lax.axis_index("c")


def _detile_kernel(vocab):
    ncol = 256
    n_it = (vocab // 128) // 2
    n_tailrows = (vocab % 128) // 2
    per_w = (n_it + _NW - 1) // _NW
    pad = 256

    @functools.partial(
        pl.kernel,
        out_type=jax.ShapeDtypeStruct((vocab // 2, 128), jnp.float32),
        mesh=plsc.VectorSubcoreMesh(**_MESH),
        scratch_types=[pltpu.VMEM((64, pad), jnp.float32),
                       pltpu.VMEM((128, 128), jnp.float32),
                       pltpu.SemaphoreType.DMA],
        compiler_params=_TC_TILED,
    )
    def k1(tab_hbm, tail_hbm, out_hbm, buf, obuf, sem):
        wid = _wid()
        iota = lax.iota(jnp.int32, _L16)

        def it_body(i, carry):
            it = wid + i * _NW

            @pl.when(it < n_it)
            def _():
                c0 = it * ncol
                pltpu.async_copy(tab_hbm.at[:, pl.ds(c0, ncol)],
                                 buf, sem).wait()

                @plsc.parallel_loop(0, 32, unroll=4)
                def row_body(r4):
                    for rr in range(4):
                        r = r4 * 4 + rr
                        for j in range(8):
                            col = 2 * r + (1 if j >= 4 else 0)
                            colv = jnp.full((_L16,), 0, jnp.int32) + col
                            dv = (j % 4) * 16 + iota
                            g = plsc.load_gather(buf, [dv, colv])
                            obuf[r, pl.ds(j * 16, 16)] = g + g
                pltpu.sync_copy(obuf, out_hbm.at[pl.ds(it * 128, 128)])
            return carry

        lax.fori_loop(0, per_w, it_body, 0)

        if n_tailrows:
            @pl.when(wid == 1 % _NW)
            def _():
                pltpu.sync_copy(tail_hbm, obuf.at[pl.ds(0, n_tailrows)])
                pltpu.sync_copy(obuf.at[pl.ds(0, n_tailrows)],
                                out_hbm.at[pl.ds(n_it * 128, n_tailrows)])

    return k1


def _detile_tc(vocab):
    tb = 512
    grid = (vocab + tb - 1) // tb

    def body(tab_ref, out_ref):
        t = tab_ref[...].T * 2.0
        out_ref[...] = jnp.concatenate([t, t], axis=-1)

    return pl.pallas_call(
        body,
        grid=(grid,),
        in_specs=[pl.BlockSpec((64, tb), lambda i: (0, i))],
        out_specs=pl.BlockSpec((tb, 128), lambda i: (i, 0)),
        out_shape=jax.ShapeDtypeStruct((vocab, 128), jnp.float32),
    )


def _gather_kernel(seq_len, batch, vocab):
    n_bc = batch // 128
    items = seq_len * n_bc
    per_w = items // _NW
    assert items % _NW == 0
    gpad = 128

    @functools.partial(
        pl.kernel,
        out_type=jax.ShapeDtypeStruct((seq_len, 64, batch), jnp.float32),
        mesh=plsc.VectorSubcoreMesh(**_MESH),
        scratch_types=[
            pltpu.VMEM((2, 1, 128), jnp.int32),
            pltpu.VMEM((128, gpad), jnp.float32),
            pltpu.VMEM((128, gpad), jnp.float32),
            pltpu.VMEM((8, 128), jnp.float32),
            pltpu.VMEM((64, 128), jnp.float32),
            pltpu.SemaphoreType.DMA,
            pltpu.SemaphoreType.DMA,
        ],
        compiler_params=_TC_TILED,
    )
    def k2(seq_hbm, tab_hbm, pe_hbm, out_hbm, rowv, gbuf0,
           gbuf1, pebuf, obuf, sem0, sem1):
        wid = _wid()
        iota = lax.iota(jnp.int32, _L16)
        gbufs = (gbuf0, gbuf1)
        sems = (sem0, sem1)

        def fire(item, par):
            l = item // n_bc
            bc = item % n_bc
            pltpu.sync_copy(seq_hbm.at[l, pl.ds(bc * 128, 128)],
                            rowv.at[par, 0])
            pltpu.async_copy(tab_hbm.at[rowv.at[par, 0]],
                             gbufs[par], sems[par])

        def consume(item, par):
            gbuf = gbufs[par]
            l = item // n_bc
            bc = item % n_bc
            pltpu.sync_copy(pe_hbm.at[l], pebuf)
            pltpu.make_async_copy(
                tab_hbm.at[rowv.at[par, 0]],
                gbuf, sems[par]).wait()

            zero16 = iota * 0
            for jb in range(8):
                sl = pl.ds(jb * 16, 16)
                blv = jb * 16 + iota

                @plsc.parallel_loop(0, 16, unroll=4)
                def d_body(d4):
                    for dd in range(4):
                        d = d4 * 4 + dd
                        pe_d = pebuf[d // 8, pl.ds((d % 8) * 16, 16)]
                        g = plsc.load_gather(gbuf, [blv, zero16 + d])
                        obuf[d, sl] = g + pe_d
            pltpu.sync_copy(obuf,
                            out_hbm.at[l, :, pl.ds(bc * 128, 128)])

        fire(wid, 0)

        def item_body(i, carry):
            item = wid + i * _NW

            @pl.when(i % 2 == 0)
            def _():
                @pl.when(i + 1 < per_w)
                def _():
                    fire(item + _NW, 1)
                consume(item, 0)

            @pl.when(i % 2 == 1)
            def _():
                @pl.when(i + 1 < per_w)
                def _():
                    fire(item + _NW, 0)
                consume(item, 1)
            return carry

        lax.fori_loop(0, per_w, item_body, 0)

    return k2


@functools.partial(jax.jit, static_argnames=("batch", "seq_len", "vocab"))
def _embed(seqT, tableT, pe2b, batch, seq_len, vocab):
    scratch = _detile_tc(vocab)(tableT)
    o3 = _gather_kernel(seq_len, batch, vocab)(seqT, scratch, pe2b)
    return jnp.transpose(o3, (2, 0, 1))


def kernel(sequence, table):
    batch, seq_len = sequence.shape
    vocab = table.shape[0]
    pe2 = _make_pe2(_MAX_LEN, _D)[:seq_len]
    pe2b = jnp.asarray(
        np.repeat(pe2, _L16, axis=1).reshape(seq_len, 8, 128))
    return _embed(sequence.T.astype(jnp.int32), table.T, pe2b,
                  batch, seq_len, vocab)


# speedup vs baseline: 1.7707x; 1.7707x over previous
import functools
import numpy as np
import jax
import jax.numpy as jnp
from jax import lax
from jax.experimental import pallas as pl
from jax.experimental.pallas import tpu as pltpu
from jax.experimental.pallas import tpu_sc as plsc

_D = 64
_MAX_LEN = 512
_NUM_CORES = 2
_NUM_SUBCORES = 16
_NW = _NUM_CORES * _NUM_SUBCORES
_LANES = 16


def _make_pe2(max_len, d_model):
    position = np.arange(max_len, dtype=np.float32)[:, None]
    div_term = np.exp(
        np.arange(0, d_model, 2, dtype=np.float32) * -(np.log(10000.0) / d_model)
    )
    pe = np.zeros((max_len, d_model), dtype=np.float32)
    pe[:, 0::2] = np.sin(position * div_term)
    pe[:, 1::2] = np.cos(position * div_term)
    return pe * 2.0


@functools.partial(jax.jit, static_argnames=("batch", "seq_len"))
def _embed(seq, pe2, table, batch, seq_len):
    chunks = []
    l0 = 0
    while l0 < seq_len:
        n = min(128, seq_len - l0)
        chunks.append((l0, n))
        l0 += n
    batches_per_w = batch // _NW

    mesh = plsc.VectorSubcoreMesh(
        core_axis_name="c", subcore_axis_name="s",
        num_cores=_NUM_CORES, num_subcores=_NUM_SUBCORES,
    )

    scratch = [pltpu.VMEM((seq_len, _D), jnp.float32)]
    for _, n in chunks:
        scratch.append(pltpu.VMEM((n,), jnp.int32))
        scratch.append(pltpu.VMEM((n, _D), jnp.float32))
    scratch.append(pltpu.SemaphoreType.DMA)

    @functools.partial(
        pl.kernel,
        out_type=jax.ShapeDtypeStruct((batch, seq_len, _D), jnp.float32),
        mesh=mesh,
        scratch_types=scratch,
        compiler_params=pltpu.CompilerParams(use_tc_tiling_on_sc=False,
                                             disable_bounds_checks=True),
    )
    def body(seq_hbm, pe2_hbm, table_hbm, out_hbm, pe2_v, *rest):
        bufs = []
        for i in range(len(chunks)):
            bufs.append((rest[2 * i], rest[2 * i + 1]))
        sem = rest[-1]
        wid = lax.axis_index("s") * _NUM_CORES + lax.axis_index("c")
        pltpu.sync_copy(pe2_hbm, pe2_v)

        def batch_body(bi, carry):
            gb = wid * batches_per_w + bi
            for (l0, n), (idx_v, row_v) in zip(chunks, bufs):
                pltpu.sync_copy(seq_hbm.at[gb, pl.ds(l0, n)], idx_v)
                pltpu.async_copy(table_hbm.at[idx_v], row_v, sem).wait()

                @plsc.parallel_loop(0, n, unroll=4)
                def row_body(r):
                    for j in range(_D // _LANES):
                        sl = pl.ds(j * _LANES, _LANES)
                        x = row_v[r, sl]
                        p = pe2_v[l0 + r, sl]
                        row_v[r, sl] = x + x + p
                pltpu.sync_copy(row_v, out_hbm.at[gb, pl.ds(l0, n)])
            return carry

        lax.fori_loop(0, batches_per_w, batch_body, 0)

    return body(seq, pe2, table)


def kernel(sequence, table):
    batch, seq_len = sequence.shape
    pe2 = jnp.asarray(_make_pe2(_MAX_LEN, _D)[:seq_len])
    return _embed(sequence.astype(jnp.int32), pe2, table, batch, seq_len)
